# Initial kernel scaffold; baseline (speedup 1.0000x reference)
import jax
import jax.numpy as jnp
from jax.experimental import pallas as pl


def kernel(input, target, W):
    raise NotImplementedError("write your pallas kernel here")


# speedup vs baseline: 4.2951x; 4.2951x over previous
import functools

import jax
import jax.numpy as jnp
from jax import lax
from jax.experimental import pallas as pl
from jax.experimental.pallas import tpu as pltpu
from jax.experimental.pallas import tpu_sc as plsc

N_NEG = 16
NC, NS = 2, 16
NW = NC * NS


def _sc_gather(table, idx):
    B, S = idx.shape
    D = table.shape[1]
    b_per_w = B // NW
    mesh = plsc.VectorSubcoreMesh(core_axis_name="c", subcore_axis_name="s")

    @functools.partial(
        pl.kernel,
        mesh=mesh,
        compiler_params=pltpu.CompilerParams(use_tc_tiling_on_sc=False),
        out_type=jax.ShapeDtypeStruct((B, S, D), table.dtype),
        scratch_types=[
            pltpu.VMEM((S,), jnp.int32),
            pltpu.VMEM((S, D), jnp.float32),
            pltpu.SemaphoreType.DMA,
        ],
    )
    def k(table_hbm, idx_hbm, out_hbm, idx_v, rows_v, sem):
        wid = lax.axis_index("s") * NC + lax.axis_index("c")

        def body(i, carry):
            b = wid * b_per_w + i
            pltpu.sync_copy(idx_hbm.at[b], idx_v)
            pltpu.async_copy(table_hbm.at[idx_v], rows_v, sem).wait()
            pltpu.sync_copy(rows_v, out_hbm.at[b])
            return carry

        lax.fori_loop(0, b_per_w, body, 0)

    return k(table, idx)


def _tc_loss(E, x, n_samples, bblk=8):
    B, L, D = x.shape

    def body(e_ref, x_ref, o_ref):
        xb = x_ref[...]
        xss = jnp.sum(xb * xb, axis=1, keepdims=True)
        xn = xb / jnp.maximum(jnp.sqrt(xss), 1e-12)
        logits = []
        for k in range(n_samples):
            Ek = e_ref[:, k * L:(k + 1) * L, :]
            ess = jnp.sum(Ek * Ek, axis=1, keepdims=True)
            rn = 1.0 / jnp.maximum(jnp.sqrt(ess), 1e-12)
            logits.append(jnp.sum(Ek * rn * xn, axis=2))
        m = logits[0]
        for lk in logits[1:]:
            m = jnp.maximum(m, lk)
        s = jnp.exp(logits[0] - m)
        for lk in logits[1:]:
            s = s + jnp.exp(lk - m)
        o_ref[...] = m + jnp.log(s) - logits[0]

    return pl.pallas_call(
        body,
        grid=(B // bblk,),
        in_specs=[
            pl.BlockSpec((bblk, n_samples * L, D), lambda b: (b, 0, 0)),
            pl.BlockSpec((bblk, L, D), lambda b: (b, 0, 0)),
        ],
        out_specs=pl.BlockSpec((bblk, L), lambda b: (b, 0)),
        out_shape=jax.ShapeDtypeStruct((B, L), jnp.float32),
    )(E, x)


def kernel(input, target, W):
    B, L, D = input.shape
    V = W.shape[0]
    neg_key = jax.random.key(42)
    neg = jax.random.randint(neg_key, (B, L, N_NEG), 0, V - 1, dtype=jnp.int32)
    neg = neg + (neg >= target[..., None]).astype(jnp.int32)
    idx = jnp.concatenate([target[..., None], neg], axis=-1)
    idx = jnp.transpose(idx, (0, 2, 1)).reshape(B, (1 + N_NEG) * L)
    E = _sc_gather(W, idx)
    return _tc_loss(E, input, 1 + N_NEG)


# speedup vs baseline: 4.5287x; 1.0544x over previous
import functools

import jax
import jax.numpy as jnp
from jax import lax
from jax.experimental import pallas as pl
from jax.experimental.pallas import tpu as pltpu
from jax.experimental.pallas import tpu_sc as plsc

N_NEG = 16
NC, NS = 2, 16
NW = NC * NS


def _sc_gather(table, idx):
    B, S = idx.shape
    D = table.shape[1]
    b_per_w = B // NW
    npairs = b_per_w // 2
    mesh = plsc.VectorSubcoreMesh(core_axis_name="c", subcore_axis_name="s")

    @functools.partial(
        pl.kernel,
        mesh=mesh,
        compiler_params=pltpu.CompilerParams(use_tc_tiling_on_sc=False),
        out_type=jax.ShapeDtypeStruct((B, S, D), table.dtype),
        scratch_types=[
            pltpu.VMEM((S,), jnp.int32),
            pltpu.VMEM((S,), jnp.int32),
            pltpu.VMEM((S, D), jnp.float32),
            pltpu.VMEM((S, D), jnp.float32),
            pltpu.SemaphoreType.DMA,
            pltpu.SemaphoreType.DMA,
        ],
    )
    def k(table_hbm, idx_hbm, out_hbm, idx0, idx1, rows0, rows1, sem0, sem1):
        wid = lax.axis_index("s") * NC + lax.axis_index("c")
        b0 = wid * b_per_w
        pltpu.sync_copy(idx_hbm.at[b0], idx0)
        pltpu.async_copy(table_hbm.at[idx0], rows0, sem0)

        def pair(p, carry):
            b = b0 + 2 * p
            pltpu.sync_copy(idx_hbm.at[b + 1], idx1)
            pltpu.async_copy(table_hbm.at[idx1], rows1, sem1)
            pltpu.make_async_copy(table_hbm.at[idx0], rows0, sem0).wait()
            pltpu.sync_copy(rows0, out_hbm.at[b])

            @pl.when(p + 1 < npairs)
            def _():
                pltpu.sync_copy(idx_hbm.at[b + 2], idx0)
                pltpu.async_copy(table_hbm.at[idx0], rows0, sem0)

            pltpu.make_async_copy(table_hbm.at[idx1], rows1, sem1).wait()
            pltpu.sync_copy(rows1, out_hbm.at[b + 1])
            return carry

        lax.fori_loop(0, npairs, pair, 0)

    return k(table, idx)


def _tc_loss(E, x, n_samples, bblk=8):
    B, L, D = x.shape

    def body(e_ref, x_ref, o_ref):
        xb = x_ref[...]
        xss = jnp.sum(xb * xb, axis=1, keepdims=True)
        xn = xb * jnp.minimum(lax.rsqrt(xss), 1e12)
        logits = []
        for k in range(n_samples):
            Ek = e_ref[:, k * L:(k + 1) * L, :]
            ess = jnp.sum(Ek * Ek, axis=1, keepdims=True)
            rn = jnp.minimum(lax.rsqrt(ess), 1e12)
            logits.append(jnp.sum(Ek * rn * xn, axis=2))
        m = logits[0]
        for lk in logits[1:]:
            m = jnp.maximum(m, lk)
        s = jnp.exp(logits[0] - m)
        for lk in logits[1:]:
            s = s + jnp.exp(lk - m)
        o_ref[...] = m + jnp.log(s) - logits[0]

    return pl.pallas_call(
        body,
        grid=(B // bblk,),
        in_specs=[
            pl.BlockSpec((bblk, n_samples * L, D), lambda b: (b, 0, 0)),
            pl.BlockSpec((bblk, L, D), lambda b: (b, 0, 0)),
        ],
        out_specs=pl.BlockSpec((bblk, L), lambda b: (b, 0)),
        out_shape=jax.ShapeDtypeStruct((B, L), jnp.float32),
    )(E, x)


def kernel(input, target, W):
    B, L, D = input.shape
    V = W.shape[0]
    neg_key = jax.random.key(42)
    neg = jax.random.randint(neg_key, (B, L, N_NEG), 0, V - 1, dtype=jnp.int32)
    neg_t = jnp.transpose(neg, (0, 2, 1))
    neg_t = neg_t + (neg_t >= target[:, None, :]).astype(jnp.int32)
    idx = jnp.concatenate([target[:, None, :], neg_t], axis=1)
    idx = idx.reshape(B, (1 + N_NEG) * L)
    E = _sc_gather(W, idx)
    return _tc_loss(E, input, 1 + N_NEG)
